# baseline (speedup 1.0000x reference)
import functools

import jax
import jax.numpy as jnp
from jax import lax
from jax.experimental import pallas as pl
from jax.experimental.pallas import tpu as pltpu
from jax.experimental.pallas import tpu_sc as plsc

_NUM_CORES = 2
_NUM_SUBCORES = 16
_NW = _NUM_CORES * _NUM_SUBCORES
_CHUNK = 1024


def _gather_body(table_hbm, idx_hbm, out_hbm, idx_v, rows_v, sem,
                 *, b_per_w, n_chunks):
    wid = lax.axis_index("s") * _NUM_CORES + lax.axis_index("c")
    base = wid * b_per_w

    @pl.loop(0, n_chunks)
    def _chunk_loop(i):
        off = base + i * _CHUNK
        pltpu.sync_copy(idx_hbm.at[pl.ds(off, _CHUNK)], idx_v)
        pltpu.async_copy(table_hbm.at[idx_v], rows_v, sem).wait()
        pltpu.sync_copy(rows_v, out_hbm.at[pl.ds(off, _CHUNK)])


def kernel(token_ids, embeddings):
    orig_shape = token_ids.shape
    flat_idx = token_ids.reshape(-1).astype(jnp.int32)
    b = flat_idx.shape[0]
    d = embeddings.shape[1]
    b_per_w = b // _NW
    n_chunks = b_per_w // _CHUNK

    mesh = plsc.VectorSubcoreMesh(core_axis_name="c", subcore_axis_name="s")
    run = pl.kernel(
        functools.partial(_gather_body, b_per_w=b_per_w, n_chunks=n_chunks),
        out_type=jax.ShapeDtypeStruct((b, d), jnp.float32),
        mesh=mesh,
        scratch_types=[
            pltpu.VMEM((_CHUNK,), jnp.int32),
            pltpu.VMEM((_CHUNK, d), jnp.float32),
            pltpu.SemaphoreType.DMA,
        ],
        compiler_params=pltpu.CompilerParams(use_tc_tiling_on_sc=False),
    )
    out = run(embeddings, flat_idx)
    return out.reshape(orig_shape + (d,))


# speedup vs baseline: 1.0161x; 1.0161x over previous
import functools

import jax
import jax.numpy as jnp
from jax import lax
from jax.experimental import pallas as pl
from jax.experimental.pallas import tpu as pltpu
from jax.experimental.pallas import tpu_sc as plsc

_NUM_CORES = 2
_NUM_SUBCORES = 16
_NW = _NUM_CORES * _NUM_SUBCORES
_CHUNK = 400
_NBUF = 4


def _gather_body(table_hbm, idx_hbm, out_hbm, *scratch, b_per_w, n_groups):
    idx_bufs = scratch[0:_NBUF]
    row_bufs = scratch[_NBUF:2 * _NBUF]
    sem_i = scratch[2 * _NBUF:3 * _NBUF]
    sem_g = scratch[3 * _NBUF:4 * _NBUF]
    sem_w = scratch[4 * _NBUF:5 * _NBUF]

    wid = lax.axis_index("s") * _NUM_CORES + lax.axis_index("c")
    base = wid * b_per_w

    def off(c):
        return base + c * _CHUNK

    def start_idxload(c, b):
        pltpu.async_copy(idx_hbm.at[pl.ds(off(c), _CHUNK)], idx_bufs[b], sem_i[b])

    def wait_idxload(c, b):
        pltpu.make_async_copy(
            idx_hbm.at[pl.ds(off(c), _CHUNK)], idx_bufs[b], sem_i[b]).wait()

    def start_write(c, b):
        pltpu.async_copy(row_bufs[b], out_hbm.at[pl.ds(off(c), _CHUNK)], sem_w[b])

    def wait_write(c, b):
        pltpu.make_async_copy(
            row_bufs[b], out_hbm.at[pl.ds(off(c), _CHUNK)], sem_w[b]).wait()

    for b in range(_NBUF):
        start_idxload(b, b)
    g_descs = []
    for b in range(_NBUF):
        wait_idxload(b, b)
        g_descs.append(pltpu.async_copy(table_hbm.at[idx_bufs[b]], row_bufs[b], sem_g[b]))
    for b in range(_NBUF):
        g_descs[b].wait()
        start_write(b, b)
        start_idxload(_NBUF + b, b)

    @pl.loop(1, n_groups - 1)
    def _group(g):
        c0 = g * _NBUF
        descs = []
        for b in range(_NBUF):
            wait_idxload(c0 + b, b)
            wait_write(c0 - _NBUF + b, b)
            descs.append(pltpu.async_copy(table_hbm.at[idx_bufs[b]], row_bufs[b], sem_g[b]))
        for b in range(_NBUF):
            descs[b].wait()
            start_write(c0 + b, b)
            start_idxload(c0 + _NBUF + b, b)

    c0 = (n_groups - 1) * _NBUF
    descs = []
    for b in range(_NBUF):
        wait_idxload(c0 + b, b)
        wait_write(c0 - _NBUF + b, b)
        descs.append(pltpu.async_copy(table_hbm.at[idx_bufs[b]], row_bufs[b], sem_g[b]))
    for b in range(_NBUF):
        descs[b].wait()
        start_write(c0 + b, b)
    for b in range(_NBUF):
        wait_write(c0 + b, b)


def kernel(token_ids, embeddings):
    orig_shape = token_ids.shape
    flat_idx = token_ids.reshape(-1).astype(jnp.int32)
    b = flat_idx.shape[0]
    d = embeddings.shape[1]
    b_per_w = b // _NW
    n_groups = b_per_w // (_CHUNK * _NBUF)

    mesh = plsc.VectorSubcoreMesh(core_axis_name="c", subcore_axis_name="s")
    run = pl.kernel(
        functools.partial(_gather_body, b_per_w=b_per_w, n_groups=n_groups),
        out_type=jax.ShapeDtypeStruct((b, d), jnp.float32),
        mesh=mesh,
        scratch_types=(
            [pltpu.VMEM((_CHUNK,), jnp.int32) for _ in range(_NBUF)]
            + [pltpu.VMEM((_CHUNK, d), jnp.float32) for _ in range(_NBUF)]
            + [pltpu.SemaphoreType.DMA for _ in range(3 * _NBUF)]
        ),
        compiler_params=pltpu.CompilerParams(use_tc_tiling_on_sc=False),
    )
    out = run(embeddings, flat_idx)
    return out.reshape(orig_shape + (d,))
